# baseline (speedup 1.0000x reference)
import functools

import jax
import jax.numpy as jnp
from jax import lax
from jax.experimental import pallas as pl
from jax.experimental.pallas import tpu as pltpu
from jax.experimental.pallas import tpu_sc as plsc

N = 10000
NP = 10240
D = 128
E = 320000
NC, NS, L = 2, 16, 16
NW = NC * NS
CH = 64
NB = 4
SG = 12
NSG = 14
NCHW = SG * NSG
SGD = 12
NSGD = 14
EPW = NCHW * CH
EPAD = EPW * NW
NCHT = EPAD // CH
TRASH = NP - N
RPT = NP // NS
EPS = 1e-5
BLK = 1024
GRID = NP // BLK

_mesh = plsc.VectorSubcoreMesh(core_axis_name="c", subcore_axis_name="s")


@functools.partial(
    pl.kernel,
    out_type=jax.ShapeDtypeStruct((NC, NP), jnp.float32),
    mesh=_mesh,
    scratch_types=[
        pltpu.VMEM_SHARED((NP,), jnp.float32),
        pltpu.VMEM((SGD, CH), jnp.int32),
        pltpu.VMEM((RPT,), jnp.float32),
        pltpu.VMEM((CH,), jnp.float32),
        pltpu.SemaphoreType.DMA,
    ],
)
def _deg_kernel(dst_hbm, out_hbm, acc1, didx_v, zbuf, ones_v, dsem):
    c = lax.axis_index("c")
    s = lax.axis_index("s")
    wid = c * NS + s
    zeros = jnp.zeros((L,), jnp.float32)
    ones = jnp.ones((L,), jnp.float32)

    def zero_body(i, _):
        zbuf[pl.ds(i * L, L)] = zeros
        return 0

    lax.fori_loop(0, RPT // L, zero_body, 0)
    for j in range(CH // L):
        ones_v[pl.ds(j * L, L)] = ones
    pltpu.sync_copy(zbuf, acc1.at[pl.ds(s * RPT, RPT)])
    plsc.subcore_barrier()

    crow0 = wid * NCHW

    def body(g, _):
        pltpu.sync_copy(dst_hbm.at[pl.ds(crow0 + g * SGD, SGD), 0], didx_v)
        for k in range(SGD):
            pltpu.async_copy(ones_v, acc1.at[didx_v.at[k]], dsem, add=True)
        for k in range(SGD):
            pltpu.make_async_copy(ones_v, acc1.at[didx_v.at[k]], dsem).wait()
        return 0

    lax.fori_loop(0, NSGD, body, 0)
    plsc.subcore_barrier()
    pltpu.sync_copy(acc1.at[pl.ds(s * RPT, RPT)],
                    out_hbm.at[c, pl.ds(s * RPT, RPT)])


@functools.partial(
    pl.kernel,
    out_type=jax.ShapeDtypeStruct((NC, NP, D), jnp.float32),
    mesh=_mesh,
    scratch_types=[
        pltpu.VMEM_SHARED((NP, D), jnp.float32),
        pltpu.VMEM((2, SG, CH), jnp.int32),
        pltpu.VMEM((2, SG, CH), jnp.int32),
        pltpu.VMEM((NB, CH, D), jnp.float32),
        pltpu.SemaphoreType.DMA,
        pltpu.SemaphoreType.DMA,
        pltpu.SemaphoreType.DMA,
        pltpu.SemaphoreType.DMA,
        pltpu.SemaphoreType.DMA,
        pltpu.SemaphoreType.DMA,
        pltpu.SemaphoreType.DMA,
        pltpu.SemaphoreType.DMA,
        pltpu.SemaphoreType.DMA,
        pltpu.SemaphoreType.DMA,
    ],
)
def _agg_kernel(h_hbm, src_hbm, dst_hbm, out_hbm, acc, sidx, didx, rows,
                gsem0, gsem1, gsem2, gsem3, isem0, isem1,
                ssem0, ssem1, ssem2, ssem3):
    c = lax.axis_index("c")
    s = lax.axis_index("s")
    wid = c * NS + s
    zeros = jnp.zeros((L,), jnp.float32)

    def zrow(i, _):
        for j in range(D // L):
            rows[0, i, pl.ds(j * L, L)] = zeros
        return 0

    lax.fori_loop(0, CH, zrow, 0)
    _zsl = [(k * CH, CH) for k in range(RPT // CH)]
    if RPT % CH:
        _zsl.append((RPT - RPT % CH, RPT % CH))

    def _init_descs(sem):
        for (o, n) in _zsl:
            yield pltpu.make_async_copy(rows.at[0, pl.ds(0, n)],
                                        acc.at[pl.ds(s * RPT + o, n)], sem)

    for dsc in _init_descs(gsem0):
        dsc.start()
    for dsc in _init_descs(gsem0):
        dsc.wait()
    plsc.subcore_barrier()

    crow0 = wid * NCHW
    gsems = (gsem0, gsem1, gsem2, gsem3)
    isems = (isem0, isem1)
    ssems = (ssem0, ssem1, ssem2, ssem3)

    def load_idx(ib, sg):
        r = crow0 + sg * SG
        pltpu.async_copy(src_hbm.at[pl.ds(r, SG), 0], sidx.at[ib], isems[ib])
        pltpu.async_copy(dst_hbm.at[pl.ds(r, SG), 0], didx.at[ib], isems[ib])

    def wait_idx(ib, sg):
        r = crow0 + sg * SG
        pltpu.make_async_copy(src_hbm.at[pl.ds(r, SG), 0], sidx.at[ib],
                              isems[ib]).wait()
        pltpu.make_async_copy(dst_hbm.at[pl.ds(r, SG), 0], didx.at[ib],
                              isems[ib]).wait()

    def process_sg(ib):
        for j in range(NB - 1):
            pltpu.async_copy(h_hbm.at[sidx.at[ib, j]], rows.at[j], gsems[j])
        for k in range(SG):
            b = k % NB
            pltpu.make_async_copy(h_hbm.at[sidx.at[ib, k]], rows.at[b],
                                  gsems[b]).wait()
            if k >= 1:
                bp = (k - 1) % NB
                pltpu.make_async_copy(rows.at[bp],
                                      acc.at[didx.at[ib, k - 1]],
                                      ssems[bp]).wait()
            if k + NB - 1 < SG:
                bn = (k + NB - 1) % NB
                pltpu.async_copy(h_hbm.at[sidx.at[ib, k + NB - 1]],
                                 rows.at[bn], gsems[bn])
            pltpu.async_copy(rows.at[b], acc.at[didx.at[ib, k]], ssems[b],
                             add=True)
        bl = (SG - 1) % NB
        pltpu.make_async_copy(rows.at[bl], acc.at[didx.at[ib, SG - 1]],
                              ssems[bl]).wait()

    load_idx(0, 0)

    def pair_body(p, _):
        sg0 = 2 * p
        wait_idx(0, sg0)
        load_idx(1, sg0 + 1)
        process_sg(0)
        wait_idx(1, sg0 + 1)

        @pl.when(p + 1 < NSG // 2)
        def _():
            load_idx(0, sg0 + 2)

        process_sg(1)
        return 0

    lax.fori_loop(0, NSG // 2, pair_body, 0)
    plsc.subcore_barrier()

    def _wb_descs():
        for (o, n) in _zsl:
            r0 = s * RPT + o
            yield pltpu.make_async_copy(acc.at[pl.ds(r0, n)],
                                        out_hbm.at[c, pl.ds(r0, n)], gsem0)

    for dsc in _wb_descs():
        dsc.start()
    for dsc in _wb_descs():
        dsc.wait()


def _dinv_of(degp_ref):
    deg = jnp.sum(degp_ref[...], axis=1, keepdims=True) + 1.0
    return lax.rsqrt(deg)


def _mm_scale_body(x_ref, w_ref, degp_ref, o_ref):
    h = jnp.dot(x_ref[...], w_ref[...], preferred_element_type=jnp.float32)
    o_ref[...] = h * _dinv_of(degp_ref)


def _mid_body(p0_ref, p1_ref, hp_ref, degp_ref, b_ref, g_ref, be_ref, w_ref,
              o_ref):
    rs = 1.0 / jnp.sqrt(1.0 + EPS)
    dinv = _dinv_of(degp_ref)
    t = (p0_ref[...] + p1_ref[...] + hp_ref[...]) * dinv + b_ref[...]
    t = t * (g_ref[...] * rs) + be_ref[...]
    t = jnp.maximum(t, 0.0)
    h = jnp.dot(t, w_ref[...], preferred_element_type=jnp.float32)
    o_ref[...] = h * dinv


def _fin_body(p0_ref, p1_ref, hp_ref, degp_ref, b_ref, o_ref):
    z = ((p0_ref[...] + p1_ref[...] + hp_ref[...]) * _dinv_of(degp_ref)
         + b_ref[...])
    m = jnp.max(z, axis=1, keepdims=True)
    ez = jnp.exp(z - m)
    o_ref[...] = z - m - jnp.log(jnp.sum(ez, axis=1, keepdims=True))


_row_spec = pl.BlockSpec((BLK, D), lambda i: (i, 0))
_deg_spec = pl.BlockSpec((BLK, NC), lambda i: (i, 0))
_vec_spec = pl.BlockSpec((1, D), lambda i: (0, 0))
_w_spec = pl.BlockSpec((D, D), lambda i: (0, 0))

_mm_scale_call = pl.pallas_call(
    _mm_scale_body,
    grid=(GRID,),
    in_specs=[_row_spec, _w_spec, _deg_spec],
    out_specs=_row_spec,
    out_shape=jax.ShapeDtypeStruct((NP, D), jnp.float32))

_mid_call = pl.pallas_call(
    _mid_body,
    grid=(GRID,),
    in_specs=[_row_spec, _row_spec, _row_spec, _deg_spec,
              _vec_spec, _vec_spec, _vec_spec, _w_spec],
    out_specs=_row_spec,
    out_shape=jax.ShapeDtypeStruct((NP, D), jnp.float32))

_FB = 1000
_fin_call = pl.pallas_call(
    _fin_body,
    grid=(N // _FB,),
    in_specs=[pl.BlockSpec((_FB, D), lambda i: (i, 0)),
              pl.BlockSpec((_FB, D), lambda i: (i, 0)),
              pl.BlockSpec((_FB, D), lambda i: (i, 0)),
              pl.BlockSpec((_FB, NC), lambda i: (i, 0)),
              pl.BlockSpec((1, D), lambda i: (0, 0))],
    out_specs=pl.BlockSpec((_FB, D), lambda i: (i, 0)),
    out_shape=jax.ShapeDtypeStruct((N, D), jnp.float32))


def kernel(x, edge_index, W1, b1, bn1_g, bn1_b, W2, b2, bn2_g, bn2_b, W3, b3):
    npad = EPAD - E
    fill = jnp.arange(npad, dtype=jnp.int32) % TRASH
    src = jnp.concatenate([edge_index[0], fill])
    dst = jnp.concatenate([edge_index[1], N + fill])
    src2 = src.reshape(NCHT, 1, CH)
    dst2 = dst.reshape(NCHT, 1, CH)
    xp = jnp.concatenate([x, jnp.zeros((NP - N, D), jnp.float32)], axis=0)

    degp = _deg_kernel(dst2).T

    b1r = b1[None, :]
    b2r = b2[None, :]
    b3r = b3[None, :]
    g1r = bn1_g[None, :]
    g2r = bn2_g[None, :]
    be1r = bn1_b[None, :]
    be2r = bn2_b[None, :]

    h1 = _mm_scale_call(xp, W1, degp)
    a1 = _agg_kernel(h1, src2, dst2)
    h2 = _mid_call(a1[0], a1[1], h1, degp, b1r, g1r, be1r, W2)
    a2 = _agg_kernel(h2, src2, dst2)
    h3 = _mid_call(a2[0], a2[1], h2, degp, b2r, g2r, be2r, W3)
    a3 = _agg_kernel(h3, src2, dst2)
    return _fin_call(a3[0], a3[1], h3, degp, b3r)


# speedup vs baseline: 1.0051x; 1.0051x over previous
import functools

import jax
import jax.numpy as jnp
from jax import lax
from jax.experimental import pallas as pl
from jax.experimental.pallas import tpu as pltpu
from jax.experimental.pallas import tpu_sc as plsc

N = 10000
NP = 10240
D = 128
E = 320000
NC, NS, L = 2, 16, 16
NW = NC * NS
CH = 96
NB = 3
SG = 8
NSG = 14
NCHW = SG * NSG
SGD = 16
NSGD = 7
EPW = NCHW * CH
EPAD = EPW * NW
NCHT = EPAD // CH
TRASH = NP - N
RPT = NP // NS
EPS = 1e-5
BLK = 1024
GRID = NP // BLK

_mesh = plsc.VectorSubcoreMesh(core_axis_name="c", subcore_axis_name="s")


@functools.partial(
    pl.kernel,
    out_type=jax.ShapeDtypeStruct((NC, NP), jnp.float32),
    mesh=_mesh,
    scratch_types=[
        pltpu.VMEM_SHARED((NP,), jnp.float32),
        pltpu.VMEM((SGD, CH), jnp.int32),
        pltpu.VMEM((RPT,), jnp.float32),
        pltpu.VMEM((CH,), jnp.float32),
        pltpu.SemaphoreType.DMA,
    ],
)
def _deg_kernel(dst_hbm, out_hbm, acc1, didx_v, zbuf, ones_v, dsem):
    c = lax.axis_index("c")
    s = lax.axis_index("s")
    wid = c * NS + s
    zeros = jnp.zeros((L,), jnp.float32)
    ones = jnp.ones((L,), jnp.float32)

    def zero_body(i, _):
        zbuf[pl.ds(i * L, L)] = zeros
        return 0

    lax.fori_loop(0, RPT // L, zero_body, 0)
    for j in range(CH // L):
        ones_v[pl.ds(j * L, L)] = ones
    pltpu.sync_copy(zbuf, acc1.at[pl.ds(s * RPT, RPT)])
    plsc.subcore_barrier()

    crow0 = wid * NCHW

    def body(g, _):
        pltpu.sync_copy(dst_hbm.at[pl.ds(crow0 + g * SGD, SGD), 0], didx_v)
        for k in range(SGD):
            pltpu.async_copy(ones_v, acc1.at[didx_v.at[k]], dsem, add=True)
        for k in range(SGD):
            pltpu.make_async_copy(ones_v, acc1.at[didx_v.at[k]], dsem).wait()
        return 0

    lax.fori_loop(0, NSGD, body, 0)
    plsc.subcore_barrier()
    pltpu.sync_copy(acc1.at[pl.ds(s * RPT, RPT)],
                    out_hbm.at[c, pl.ds(s * RPT, RPT)])


@functools.partial(
    pl.kernel,
    out_type=jax.ShapeDtypeStruct((NC, NP, D), jnp.float32),
    mesh=_mesh,
    scratch_types=[
        pltpu.VMEM_SHARED((NP, D), jnp.float32),
        pltpu.VMEM((2, SG, CH), jnp.int32),
        pltpu.VMEM((2, SG, CH), jnp.int32),
        pltpu.VMEM((NB, CH, D), jnp.float32),
        pltpu.SemaphoreType.DMA,
        pltpu.SemaphoreType.DMA,
        pltpu.SemaphoreType.DMA,
        pltpu.SemaphoreType.DMA,
        pltpu.SemaphoreType.DMA,
        pltpu.SemaphoreType.DMA,
        pltpu.SemaphoreType.DMA,
        pltpu.SemaphoreType.DMA,
    ],
)
def _agg_kernel(h_hbm, src_hbm, dst_hbm, out_hbm, acc, sidx, didx, rows,
                gsem0, gsem1, gsem2, isem0, isem1, ssem0, ssem1, ssem2):
    c = lax.axis_index("c")
    s = lax.axis_index("s")
    wid = c * NS + s
    zeros = jnp.zeros((L,), jnp.float32)

    def zrow(i, _):
        for j in range(D // L):
            rows[0, i, pl.ds(j * L, L)] = zeros
        return 0

    lax.fori_loop(0, CH, zrow, 0)
    _zsl = [(k * CH, CH) for k in range(RPT // CH)]
    if RPT % CH:
        _zsl.append((RPT - RPT % CH, RPT % CH))

    def _init_descs(sem):
        for (o, n) in _zsl:
            yield pltpu.make_async_copy(rows.at[0, pl.ds(0, n)],
                                        acc.at[pl.ds(s * RPT + o, n)], sem)

    for dsc in _init_descs(gsem0):
        dsc.start()
    for dsc in _init_descs(gsem0):
        dsc.wait()
    plsc.subcore_barrier()

    crow0 = wid * NCHW
    gsems = (gsem0, gsem1, gsem2)
    isems = (isem0, isem1)
    ssems = (ssem0, ssem1, ssem2)

    def load_idx(ib, sg):
        r = crow0 + sg * SG
        pltpu.async_copy(src_hbm.at[pl.ds(r, SG), 0], sidx.at[ib], isems[ib])
        pltpu.async_copy(dst_hbm.at[pl.ds(r, SG), 0], didx.at[ib], isems[ib])

    def wait_idx(ib, sg):
        r = crow0 + sg * SG
        pltpu.make_async_copy(src_hbm.at[pl.ds(r, SG), 0], sidx.at[ib],
                              isems[ib]).wait()
        pltpu.make_async_copy(dst_hbm.at[pl.ds(r, SG), 0], didx.at[ib],
                              isems[ib]).wait()

    def process_sg(ib):
        for j in range(NB - 1):
            pltpu.async_copy(h_hbm.at[sidx.at[ib, j]], rows.at[j], gsems[j])
        for k in range(SG):
            b = k % NB
            pltpu.make_async_copy(h_hbm.at[sidx.at[ib, k]], rows.at[b],
                                  gsems[b]).wait()
            if k >= 1:
                bp = (k - 1) % NB
                pltpu.make_async_copy(rows.at[bp],
                                      acc.at[didx.at[ib, k - 1]],
                                      ssems[bp]).wait()
            if k + NB - 1 < SG:
                bn = (k + NB - 1) % NB
                pltpu.async_copy(h_hbm.at[sidx.at[ib, k + NB - 1]],
                                 rows.at[bn], gsems[bn])
            pltpu.async_copy(rows.at[b], acc.at[didx.at[ib, k]], ssems[b],
                             add=True)
        bl = (SG - 1) % NB
        pltpu.make_async_copy(rows.at[bl], acc.at[didx.at[ib, SG - 1]],
                              ssems[bl]).wait()

    load_idx(0, 0)

    def pair_body(p, _):
        sg0 = 2 * p
        wait_idx(0, sg0)
        load_idx(1, sg0 + 1)
        process_sg(0)
        wait_idx(1, sg0 + 1)

        @pl.when(p + 1 < NSG // 2)
        def _():
            load_idx(0, sg0 + 2)

        process_sg(1)
        return 0

    lax.fori_loop(0, NSG // 2, pair_body, 0)
    plsc.subcore_barrier()

    def _wb_descs():
        for (o, n) in _zsl:
            r0 = s * RPT + o
            yield pltpu.make_async_copy(acc.at[pl.ds(r0, n)],
                                        out_hbm.at[c, pl.ds(r0, n)], gsem0)

    for dsc in _wb_descs():
        dsc.start()
    for dsc in _wb_descs():
        dsc.wait()


def _dinv_of(degp_ref):
    deg = jnp.sum(degp_ref[...], axis=1, keepdims=True) + 1.0
    return lax.rsqrt(deg)


def _mm_scale_body(x_ref, w_ref, degp_ref, o_ref):
    h = jnp.dot(x_ref[...], w_ref[...], preferred_element_type=jnp.float32)
    o_ref[...] = h * _dinv_of(degp_ref)


def _mid_body(p0_ref, p1_ref, hp_ref, degp_ref, b_ref, g_ref, be_ref, w_ref,
              o_ref):
    rs = 1.0 / jnp.sqrt(1.0 + EPS)
    dinv = _dinv_of(degp_ref)
    t = (p0_ref[...] + p1_ref[...] + hp_ref[...]) * dinv + b_ref[...]
    t = t * (g_ref[...] * rs) + be_ref[...]
    t = jnp.maximum(t, 0.0)
    h = jnp.dot(t, w_ref[...], preferred_element_type=jnp.float32)
    o_ref[...] = h * dinv


def _fin_body(p0_ref, p1_ref, hp_ref, degp_ref, b_ref, o_ref):
    z = ((p0_ref[...] + p1_ref[...] + hp_ref[...]) * _dinv_of(degp_ref)
         + b_ref[...])
    m = jnp.max(z, axis=1, keepdims=True)
    ez = jnp.exp(z - m)
    o_ref[...] = z - m - jnp.log(jnp.sum(ez, axis=1, keepdims=True))


_row_spec = pl.BlockSpec((BLK, D), lambda i: (i, 0))
_deg_spec = pl.BlockSpec((BLK, NC), lambda i: (i, 0))
_vec_spec = pl.BlockSpec((1, D), lambda i: (0, 0))
_w_spec = pl.BlockSpec((D, D), lambda i: (0, 0))

_mm_scale_call = pl.pallas_call(
    _mm_scale_body,
    grid=(GRID,),
    in_specs=[_row_spec, _w_spec, _deg_spec],
    out_specs=_row_spec,
    out_shape=jax.ShapeDtypeStruct((NP, D), jnp.float32))

_mid_call = pl.pallas_call(
    _mid_body,
    grid=(GRID,),
    in_specs=[_row_spec, _row_spec, _row_spec, _deg_spec,
              _vec_spec, _vec_spec, _vec_spec, _w_spec],
    out_specs=_row_spec,
    out_shape=jax.ShapeDtypeStruct((NP, D), jnp.float32))

_FB = 1000
_fin_call = pl.pallas_call(
    _fin_body,
    grid=(N // _FB,),
    in_specs=[pl.BlockSpec((_FB, D), lambda i: (i, 0)),
              pl.BlockSpec((_FB, D), lambda i: (i, 0)),
              pl.BlockSpec((_FB, D), lambda i: (i, 0)),
              pl.BlockSpec((_FB, NC), lambda i: (i, 0)),
              pl.BlockSpec((1, D), lambda i: (0, 0))],
    out_specs=pl.BlockSpec((_FB, D), lambda i: (i, 0)),
    out_shape=jax.ShapeDtypeStruct((N, D), jnp.float32))


def kernel(x, edge_index, W1, b1, bn1_g, bn1_b, W2, b2, bn2_g, bn2_b, W3, b3):
    npad = EPAD - E
    fill = jnp.arange(npad, dtype=jnp.int32) % TRASH
    src = jnp.concatenate([edge_index[0], fill])
    dst = jnp.concatenate([edge_index[1], N + fill])
    src2 = src.reshape(NCHT, 1, CH)
    dst2 = dst.reshape(NCHT, 1, CH)
    xp = jnp.concatenate([x, jnp.zeros((NP - N, D), jnp.float32)], axis=0)

    degp = _deg_kernel(dst2).T

    b1r = b1[None, :]
    b2r = b2[None, :]
    b3r = b3[None, :]
    g1r = bn1_g[None, :]
    g2r = bn2_g[None, :]
    be1r = bn1_b[None, :]
    be2r = bn2_b[None, :]

    h1 = _mm_scale_call(xp, W1, degp)
    a1 = _agg_kernel(h1, src2, dst2)
    h2 = _mid_call(a1[0], a1[1], h1, degp, b1r, g1r, be1r, W2)
    a2 = _agg_kernel(h2, src2, dst2)
    h3 = _mid_call(a2[0], a2[1], h2, degp, b2r, g2r, be2r, W3)
    a3 = _agg_kernel(h3, src2, dst2)
    return _fin_call(a3[0], a3[1], h3, degp, b3r)
